# baseline (speedup 1.0000x reference)
import functools

import jax
import jax.numpy as jnp
from jax import lax
from jax.experimental import pallas as pl
from jax.experimental.pallas import tpu as pltpu
from jax.experimental.pallas import tpu_sc as plsc

_DIM = 64
_TAB = 4096
_B = 16384

_CHUNK = 128


def _prep_body(xr_ref, table_ref, idx_ref):
    xv = xr_ref[...]
    ml = jnp.floor(jnp.max(xv)) + jnp.float32(1.0)
    idx_ref[...] = ((xv / ml) * (ml - jnp.float32(1.0))).astype(jnp.int32)
    c = lax.broadcasted_iota(jnp.int32, (_TAB // 2, 2 * _DIM), 1)
    m = lax.broadcasted_iota(jnp.int32, (_TAB // 2, 2 * _DIM), 0)
    pos = (2 * m + (c // _DIM)).astype(jnp.float32)
    col = c % _DIM
    dt = jnp.exp((col & ~1).astype(jnp.float32) * (-jnp.log(ml) / _DIM))
    ang = pos * dt
    table_ref[...] = jnp.where((col & 1) == 0, jnp.sin(ang), jnp.cos(ang))


_prep = pl.pallas_call(
    _prep_body,
    out_shape=(
        jax.ShapeDtypeStruct((_TAB // 2, 2 * _DIM), jnp.float32),
        jax.ShapeDtypeStruct((_B * 2 // 128, 128), jnp.int32),
    ),
)


@functools.cache
def _make_gather_add():
    info = plsc.get_sparse_core_info()
    nc, ns = info.num_cores, info.num_subcores
    nw = nc * ns
    rpw = _B // nw
    nblk = rpw // _CHUNK

    @functools.partial(
        pl.kernel,
        mesh=plsc.VectorSubcoreMesh(core_axis_name="c", subcore_axis_name="s"),
        out_type=jax.ShapeDtypeStruct((_B, _DIM), jnp.float32),
        scratch_types=[
            pltpu.VMEM((2 * nblk, _CHUNK), jnp.int32),
            pltpu.VMEM((rpw, _DIM), jnp.float32),
            [pltpu.SemaphoreType.DMA] * 4,
            [pltpu.SemaphoreType.DMA] * 4,
            pltpu.SemaphoreType.DMA,
        ],
        compiler_params=pltpu.CompilerParams(use_tc_tiling_on_sc=False),
    )
    def _gather_add(table_hbm, idx_hbm, out_hbm, idx_v, g_v, gsems, asems, osem):
        wid = lax.axis_index("s") * nc + lax.axis_index("c")
        pltpu.sync_copy(idx_hbm.at[pl.ds(wid * 2 * nblk, 2 * nblk)], idx_v)
        gathers = [
            pltpu.async_copy(
                table_hbm.at[idx_v.at[2 * t]],
                g_v.at[pl.ds(t * _CHUNK, _CHUNK)],
                gsems[t],
            )
            for t in range(nblk)
        ]
        base = wid * rpw
        adds = []
        writes = []
        for t in range(nblk):
            gathers[t].wait()
            adds.append(
                pltpu.async_copy(
                    table_hbm.at[idx_v.at[2 * t + 1]],
                    g_v.at[pl.ds(t * _CHUNK, _CHUNK)],
                    asems[t],
                    add=True,
                )
            )
        for t in range(nblk):
            adds[t].wait()
            writes.append(
                pltpu.async_copy(
                    g_v.at[pl.ds(t * _CHUNK, _CHUNK)],
                    out_hbm.at[pl.ds(base + t * _CHUNK, _CHUNK)],
                    osem,
                )
            )
        for wr in writes:
            wr.wait()

    return _gather_add


def _finish_body(v_ref, o_ref):
    v = v_ref[...]
    at = v[:, 0:_DIM].T
    bt = v[:, _DIM:2 * _DIM].T
    o_ref[...] = jnp.stack([at, bt], axis=-1).reshape(_DIM, 1024)


_finish = pl.pallas_call(
    _finish_body,
    grid=(16,),
    in_specs=[pl.BlockSpec((512, 128), lambda g: (g, 0))],
    out_specs=pl.BlockSpec((_DIM, 1024), lambda g: (0, g)),
    out_shape=jax.ShapeDtypeStruct((_DIM, _B), jnp.float32),
)


def kernel(x):
    y = x.reshape(128, 128, 2).transpose(0, 2, 1).reshape(_B * 2 // 128, 128)
    table2, idx = _prep(y)
    table = table2.reshape(_TAB, _DIM)
    s = _make_gather_add()(table, idx)
    out64 = _finish(s.reshape(_B // 2, 2 * _DIM))
    return out64.T


# speedup vs baseline: 5.9446x; 5.9446x over previous
import functools

import jax
import jax.numpy as jnp
from jax import lax
from jax.experimental import pallas as pl
from jax.experimental.pallas import tpu as pltpu
from jax.experimental.pallas import tpu_sc as plsc

_DIM = 64
_TAB = 4096
_B = 16384

_CHUNK = 128


def _prep_body(xr_ref, table_ref, idx_ref):
    xv = xr_ref[...]
    ml = jnp.floor(jnp.max(xv)) + jnp.float32(1.0)
    idx_ref[...] = ((xv / ml) * (ml - jnp.float32(1.0))).astype(jnp.int32)
    c = lax.broadcasted_iota(jnp.int32, (_TAB // 2, 2 * _DIM), 1)
    m = lax.broadcasted_iota(jnp.int32, (_TAB // 2, 2 * _DIM), 0)
    pos = (2 * m + (c // _DIM)).astype(jnp.float32)
    col = c % _DIM
    dt = jnp.exp((col & ~1).astype(jnp.float32) * (-jnp.log(ml) / _DIM))
    ang = pos * dt
    table_ref[...] = jnp.where((col & 1) == 0, jnp.sin(ang), jnp.cos(ang))


_prep = pl.pallas_call(
    _prep_body,
    out_shape=(
        jax.ShapeDtypeStruct((_TAB // 2, 2 * _DIM), jnp.float32),
        jax.ShapeDtypeStruct((_B * 2 // 128, 128), jnp.int32),
    ),
)


@functools.cache
def _make_gather_add():
    info = plsc.get_sparse_core_info()
    nc, ns = info.num_cores, info.num_subcores
    nw = nc * ns
    rpw = _B // nw
    nblk = rpw // _CHUNK

    @functools.partial(
        pl.kernel,
        mesh=plsc.VectorSubcoreMesh(core_axis_name="c", subcore_axis_name="s"),
        out_type=jax.ShapeDtypeStruct((_B, _DIM), jnp.float32),
        scratch_types=[
            pltpu.VMEM((2 * nblk, _CHUNK), jnp.int32),
            pltpu.VMEM((rpw, _DIM), jnp.float32),
            [pltpu.SemaphoreType.DMA] * 4,
            [pltpu.SemaphoreType.DMA] * 4,
            pltpu.SemaphoreType.DMA,
        ],
        compiler_params=pltpu.CompilerParams(use_tc_tiling_on_sc=False),
    )
    def _gather_add(table_hbm, idx_hbm, out_hbm, idx_v, g_v, gsems, asems, osem):
        wid = lax.axis_index("s") * nc + lax.axis_index("c")
        pltpu.sync_copy(idx_hbm.at[pl.ds(wid * 2 * nblk, 2 * nblk)], idx_v)
        gathers = [
            pltpu.async_copy(
                table_hbm.at[idx_v.at[2 * t]],
                g_v.at[pl.ds(t * _CHUNK, _CHUNK)],
                gsems[t],
            )
            for t in range(nblk)
        ]
        base = wid * rpw
        adds = []
        writes = []
        for t in range(nblk):
            gathers[t].wait()
            adds.append(
                pltpu.async_copy(
                    table_hbm.at[idx_v.at[2 * t + 1]],
                    g_v.at[pl.ds(t * _CHUNK, _CHUNK)],
                    asems[t],
                    add=True,
                )
            )
        for t in range(nblk):
            adds[t].wait()
            writes.append(
                pltpu.async_copy(
                    g_v.at[pl.ds(t * _CHUNK, _CHUNK)],
                    out_hbm.at[pl.ds(base + t * _CHUNK, _CHUNK)],
                    osem,
                )
            )
        for wr in writes:
            wr.wait()

    return _gather_add


def _finish_body(v_ref, o_ref):
    v = v_ref[...]
    p = lax.broadcasted_iota(jnp.int32, (512, 1024), 0)
    m = lax.broadcasted_iota(jnp.int32, (512, 1024), 1)
    p0 = (m == 2 * p).astype(jnp.float32)
    r1 = lax.dot_general(v, p0, (((0,), (0,)), ((), ())),
                         preferred_element_type=jnp.float32)
    r2 = r1[_DIM:2 * _DIM]
    r2r = jnp.concatenate([r2[:, 1023:1024], r2[:, 0:1023]], axis=1)
    o_ref[...] = r1[0:_DIM] + r2r


_finish = pl.pallas_call(
    _finish_body,
    grid=(16,),
    in_specs=[pl.BlockSpec((512, 128), lambda g: (g, 0))],
    out_specs=pl.BlockSpec((_DIM, 1024), lambda g: (0, g)),
    out_shape=jax.ShapeDtypeStruct((_DIM, _B), jnp.float32),
)


def kernel(x):
    y = x.reshape(128, 128, 2).transpose(0, 2, 1).reshape(_B * 2 // 128, 128)
    table2, idx = _prep(y)
    table = table2.reshape(_TAB, _DIM)
    s = _make_gather_add()(table, idx)
    out64 = _finish(s.reshape(_B // 2, 2 * _DIM))
    return out64.T


# speedup vs baseline: 7.1805x; 1.2079x over previous
import functools

import jax
import jax.numpy as jnp
from jax import lax
from jax.experimental import pallas as pl
from jax.experimental.pallas import tpu as pltpu
from jax.experimental.pallas import tpu_sc as plsc

_DIM = 64
_TAB = 4096
_B = 16384

_CHUNK = 128


def _prep_body(xr_ref, table_ref, idx_ref):
    xv = xr_ref[...]
    ml = jnp.floor(jnp.max(xv)) + jnp.float32(1.0)
    idx_ref[...] = ((xv / ml) * (ml - jnp.float32(1.0))).astype(jnp.int32)
    c = lax.broadcasted_iota(jnp.int32, (_TAB // 2, 2 * _DIM), 1)
    m = lax.broadcasted_iota(jnp.int32, (_TAB // 2, 2 * _DIM), 0)
    pos = (2 * m + (c // _DIM)).astype(jnp.float32)
    col = c % _DIM
    dt = jnp.exp((col & ~1).astype(jnp.float32) * (-jnp.log(ml) / _DIM))
    ang = pos * dt + jnp.where((col & 1) == 0, 0.0, jnp.float32(jnp.pi / 2))
    table_ref[...] = jnp.sin(ang)


_prep = pl.pallas_call(
    _prep_body,
    out_shape=(
        jax.ShapeDtypeStruct((_TAB // 2, 2 * _DIM), jnp.float32),
        jax.ShapeDtypeStruct((_B * 2 // 128, 128), jnp.int32),
    ),
)


@functools.cache
def _make_gather_add():
    info = plsc.get_sparse_core_info()
    nc, ns = info.num_cores, info.num_subcores
    nw = nc * ns
    rpw = _B // nw
    nblk = rpw // _CHUNK

    @functools.partial(
        pl.kernel,
        mesh=plsc.VectorSubcoreMesh(core_axis_name="c", subcore_axis_name="s"),
        out_type=jax.ShapeDtypeStruct((_B, _DIM), jnp.float32),
        scratch_types=[
            pltpu.VMEM((2 * nblk, _CHUNK), jnp.int32),
            pltpu.VMEM((rpw, _DIM), jnp.float32),
            [pltpu.SemaphoreType.DMA] * 4,
            [pltpu.SemaphoreType.DMA] * 4,
            pltpu.SemaphoreType.DMA,
        ],
        compiler_params=pltpu.CompilerParams(use_tc_tiling_on_sc=False),
    )
    def _gather_add(table_hbm, idx_hbm, out_hbm, idx_v, g_v, gsems, asems, osem):
        wid = lax.axis_index("s") * nc + lax.axis_index("c")
        pltpu.sync_copy(idx_hbm.at[pl.ds(wid * 2 * nblk, 2 * nblk)], idx_v)
        gathers = [
            pltpu.async_copy(
                table_hbm.at[idx_v.at[2 * t]],
                g_v.at[pl.ds(t * _CHUNK, _CHUNK)],
                gsems[t],
            )
            for t in range(nblk)
        ]
        base = wid * rpw
        adds = []
        writes = []
        for t in range(nblk):
            gathers[t].wait()
            adds.append(
                pltpu.async_copy(
                    table_hbm.at[idx_v.at[2 * t + 1]],
                    g_v.at[pl.ds(t * _CHUNK, _CHUNK)],
                    asems[t],
                    add=True,
                )
            )
        for t in range(nblk):
            adds[t].wait()
            writes.append(
                pltpu.async_copy(
                    g_v.at[pl.ds(t * _CHUNK, _CHUNK)],
                    out_hbm.at[pl.ds(base + t * _CHUNK, _CHUNK)],
                    osem,
                )
            )
        for wr in writes:
            wr.wait()

    return _gather_add


_FR = 512
_FSUB = 8


def _finish_body(v_ref, o_ref, p0_ref):
    @pl.when(pl.program_id(0) == 0)
    def _():
        p = lax.broadcasted_iota(jnp.int32, (_FR, 2 * _FR), 0)
        m = lax.broadcasted_iota(jnp.int32, (_FR, 2 * _FR), 1)
        p0_ref[...] = (m == 2 * p).astype(jnp.float32)

    p0 = p0_ref[...]
    for h in range(_FSUB):
        v = v_ref[pl.ds(h * _FR, _FR), :]
        r1 = lax.dot_general(v, p0, (((0,), (0,)), ((), ())),
                             preferred_element_type=jnp.float32)
        r2 = r1[_DIM:2 * _DIM]
        r2r = jnp.concatenate([r2[:, 2 * _FR - 1:], r2[:, 0:2 * _FR - 1]],
                              axis=1)
        o_ref[:, pl.ds(h * 2 * _FR, 2 * _FR)] = r1[0:_DIM] + r2r


_finish = pl.pallas_call(
    _finish_body,
    grid=(_B // 2 // (_FR * _FSUB),),
    in_specs=[pl.BlockSpec((_FR * _FSUB, 128), lambda g: (g, 0))],
    out_specs=pl.BlockSpec((_DIM, 2 * _FR * _FSUB), lambda g: (0, g)),
    out_shape=jax.ShapeDtypeStruct((_DIM, _B), jnp.float32),
    scratch_shapes=[pltpu.VMEM((_FR, 2 * _FR), jnp.float32)],
)


def kernel(x):
    y = x.reshape(128, 128, 2).transpose(0, 2, 1).reshape(_B * 2 // 128, 128)
    table2, idx = _prep(y)
    table = table2.reshape(_TAB, _DIM)
    s = _make_gather_add()(table, idx)
    out64 = _finish(s.reshape(_B // 2, 2 * _DIM))
    return out64.T
